# baseline (speedup 1.0000x reference)
import functools

import jax
import jax.numpy as jnp
from jax import lax
from jax.experimental import pallas as pl
from jax.experimental.pallas import tpu as pltpu
from jax.experimental.pallas import tpu_sc as plsc

_VOCAB = 1000000
_EMB = 16
_CLS = 2

_ROWS_PER_BLK = 32768


def _project_body(e_ref, w_ref, b_ref, o0_ref, o1_ref):
    res = lax.dot_general(
        w_ref[...], e_ref[...],
        dimension_numbers=(((1,), (1,)), ((), ())),
        preferred_element_type=jnp.float32,
    ) + b_ref[...]
    o0_ref[...] = res[0, :]
    o1_ref[...] = res[1, :]


def _project_table(embed, W, b2d):
    grid = pl.cdiv(_VOCAB, _ROWS_PER_BLK)
    return pl.pallas_call(
        _project_body,
        grid=(grid,),
        in_specs=[
            pl.BlockSpec((_ROWS_PER_BLK, _EMB), lambda i: (i, 0)),
            pl.BlockSpec((_CLS, _EMB), lambda i: (0, 0)),
            pl.BlockSpec((_CLS, 1), lambda i: (0, 0)),
        ],
        out_specs=[
            pl.BlockSpec((_ROWS_PER_BLK,), lambda i: (i,)),
            pl.BlockSpec((_ROWS_PER_BLK,), lambda i: (i,)),
        ],
        out_shape=[
            jax.ShapeDtypeStruct((_VOCAB,), jnp.float32),
            jax.ShapeDtypeStruct((_VOCAB,), jnp.float32),
        ],
    )(embed, W, b2d)


_CH_ROWS = 2


def _make_gather(n_rows_x, row_len):
    info = plsc.get_sparse_core_info()
    nc, ns = info.num_cores, info.num_subcores
    nw = nc * ns
    rows_per_w = n_rows_x // nw
    n_total = n_rows_x * row_len
    per_w = n_total // nw
    steps = rows_per_w // _CH_ROWS
    v_stage = _VOCAB // (ns // 2)
    mesh = plsc.VectorSubcoreMesh(core_axis_name="c", subcore_axis_name="s")

    @functools.partial(
        pl.kernel,
        out_type=[
            jax.ShapeDtypeStruct((n_rows_x, row_len), jnp.float32),
            jax.ShapeDtypeStruct((n_rows_x, row_len), jnp.float32),
        ],
        mesh=mesh,
        scratch_types=[
            pltpu.VMEM_SHARED((_VOCAB,), jnp.float32),
            pltpu.VMEM_SHARED((_VOCAB,), jnp.float32),
            pltpu.VMEM((_CH_ROWS, row_len), jnp.int32),
            pltpu.VMEM((_CH_ROWS, row_len), jnp.float32),
            pltpu.VMEM((_CH_ROWS, row_len), jnp.float32),
            pltpu.SemaphoreType.DMA,
        ],
        compiler_params=pltpu.CompilerParams(use_tc_tiling_on_sc=False,
                                             needs_layout_passes=False),
    )
    def gather_kernel(p0_hbm, p1_hbm, idx_hbm, o0_hbm, o1_hbm,
                      sh0, sh1, idx_v, v0, v1, sem):
        cid = lax.axis_index("c")
        sid = lax.axis_index("s")
        wid = sid * nc + cid
        row_base = wid * rows_per_w

        @pl.when(sid < ns // 2)
        def _stage0():
            pltpu.sync_copy(p0_hbm.at[pl.ds(sid * v_stage, v_stage)],
                            sh0.at[pl.ds(sid * v_stage, v_stage)])

        @pl.when(sid >= ns // 2)
        def _stage1():
            s2 = sid - ns // 2
            pltpu.sync_copy(p1_hbm.at[pl.ds(s2 * v_stage, v_stage)],
                            sh1.at[pl.ds(s2 * v_stage, v_stage)])

        plsc.subcore_barrier()

        def step(j, carry):
            row = row_base + j * _CH_ROWS
            pltpu.sync_copy(idx_hbm.at[pl.ds(row, _CH_ROWS)], idx_v)
            copies = []
            for r in range(_CH_ROWS):
                copies.append(
                    pltpu.async_copy(sh0.at[idx_v.at[r]], v0.at[r], sem))
                copies.append(
                    pltpu.async_copy(sh1.at[idx_v.at[r]], v1.at[r], sem))
            for c in copies:
                c.wait()
            pltpu.sync_copy(v0, o0_hbm.at[pl.ds(row, _CH_ROWS)])
            pltpu.sync_copy(v1, o1_hbm.at[pl.ds(row, _CH_ROWS)])
            return carry

        lax.fori_loop(0, steps, step, 0)

    return gather_kernel


_XROWS_PER_BLK = 2048


def _passthru_body(x_ref, o_ref):
    o_ref[...] = x_ref[...]


def _passthru_x(x):
    B, L = x.shape
    grid = B // _XROWS_PER_BLK
    return pl.pallas_call(
        _passthru_body,
        grid=(grid,),
        in_specs=[pl.BlockSpec((_XROWS_PER_BLK, L), lambda i: (i, 0))],
        out_specs=pl.BlockSpec((_XROWS_PER_BLK, L), lambda i: (i, 0)),
        out_shape=jax.ShapeDtypeStruct((B, L), jnp.int32),
    )(x)


def kernel(x, embed, W, b):
    B, L = x.shape
    p0, p1 = _project_table(embed, W, b.reshape(_CLS, 1))
    o0, o1 = _make_gather(B, L)(p0, p1, _passthru_x(x))
    return jnp.stack([o0, o1], axis=-1)


# speedup vs baseline: 1.1114x; 1.1114x over previous
import functools

import jax
import jax.numpy as jnp
from jax import lax
from jax.experimental import pallas as pl
from jax.experimental.pallas import tpu as pltpu
from jax.experimental.pallas import tpu_sc as plsc

_VOCAB = 1000000
_EMB = 16
_CLS = 2

_ROWS_PER_BLK = 32768


def _project_body(e_ref, w_ref, b_ref, o0_ref, o1_ref):
    res = lax.dot_general(
        w_ref[...], e_ref[...],
        dimension_numbers=(((1,), (1,)), ((), ())),
        preferred_element_type=jnp.float32,
    ) + b_ref[...]
    o0_ref[...] = res[0, :]
    o1_ref[...] = res[1, :]


def _project_table(embed, W, b2d):
    grid = pl.cdiv(_VOCAB, _ROWS_PER_BLK)
    return pl.pallas_call(
        _project_body,
        grid=(grid,),
        in_specs=[
            pl.BlockSpec((_ROWS_PER_BLK, _EMB), lambda i: (i, 0)),
            pl.BlockSpec((_CLS, _EMB), lambda i: (0, 0)),
            pl.BlockSpec((_CLS, 1), lambda i: (0, 0)),
        ],
        out_specs=[
            pl.BlockSpec((_ROWS_PER_BLK,), lambda i: (i,)),
            pl.BlockSpec((_ROWS_PER_BLK,), lambda i: (i,)),
        ],
        out_shape=[
            jax.ShapeDtypeStruct((_VOCAB,), jnp.float32),
            jax.ShapeDtypeStruct((_VOCAB,), jnp.float32),
        ],
    )(embed, W, b2d)


_CHUNK = 640


def _make_gather(n_total):
    info = plsc.get_sparse_core_info()
    nc, ns = info.num_cores, info.num_subcores
    nw = nc * ns
    per_w = n_total // nw
    steps = per_w // _CHUNK
    v_stage = _VOCAB // (ns // 2)
    mesh = plsc.VectorSubcoreMesh(core_axis_name="c", subcore_axis_name="s")

    @functools.partial(
        pl.kernel,
        out_type=[
            jax.ShapeDtypeStruct((n_total,), jnp.float32),
            jax.ShapeDtypeStruct((n_total,), jnp.float32),
        ],
        mesh=mesh,
        scratch_types=[
            pltpu.VMEM_SHARED((_VOCAB,), jnp.float32),
            pltpu.VMEM_SHARED((_VOCAB,), jnp.float32),
            pltpu.VMEM((_CHUNK,), jnp.int32),
            pltpu.VMEM((_CHUNK,), jnp.float32),
            pltpu.VMEM((_CHUNK,), jnp.float32),
            pltpu.SemaphoreType.DMA,
        ],
        compiler_params=pltpu.CompilerParams(use_tc_tiling_on_sc=False,
                                             needs_layout_passes=False),
    )
    def gather_kernel(p0_hbm, p1_hbm, idx_hbm, o0_hbm, o1_hbm,
                      sh0, sh1, idx_v, v0, v1, sem):
        cid = lax.axis_index("c")
        sid = lax.axis_index("s")
        wid = sid * nc + cid
        base = wid * per_w

        @pl.when(sid < ns // 2)
        def _stage0():
            pltpu.sync_copy(p0_hbm.at[pl.ds(sid * v_stage, v_stage)],
                            sh0.at[pl.ds(sid * v_stage, v_stage)])

        @pl.when(sid >= ns // 2)
        def _stage1():
            s2 = sid - ns // 2
            pltpu.sync_copy(p1_hbm.at[pl.ds(s2 * v_stage, v_stage)],
                            sh1.at[pl.ds(s2 * v_stage, v_stage)])

        plsc.subcore_barrier()

        def step(j, carry):
            off = base + j * _CHUNK
            pltpu.sync_copy(idx_hbm.at[pl.ds(off, _CHUNK)], idx_v)
            c0 = pltpu.async_copy(sh0.at[idx_v], v0, sem)
            c1 = pltpu.async_copy(sh1.at[idx_v], v1, sem)
            c0.wait()
            c1.wait()
            pltpu.sync_copy(v0, o0_hbm.at[pl.ds(off, _CHUNK)])
            pltpu.sync_copy(v1, o1_hbm.at[pl.ds(off, _CHUNK)])
            return carry

        lax.fori_loop(0, steps, step, 0)

    return gather_kernel


def kernel(x, embed, W, b):
    B, L = x.shape
    p0, p1 = _project_table(embed, W, b.reshape(_CLS, 1))
    o0, o1 = _make_gather(B * L)(p0, p1, x.reshape(-1))
    return jnp.stack([o0, o1], axis=-1).reshape(B, L, _CLS)


# speedup vs baseline: 1.1487x; 1.0336x over previous
import functools

import jax
import jax.numpy as jnp
from jax import lax
from jax.experimental import pallas as pl
from jax.experimental.pallas import tpu as pltpu
from jax.experimental.pallas import tpu_sc as plsc

_VOCAB = 1000000
_EMB = 16
_CLS = 2

_ROWS_PER_BLK = 32768


def _project_body(e_ref, w_ref, b_ref, o0_ref, o1_ref):
    res = lax.dot_general(
        w_ref[...], e_ref[...],
        dimension_numbers=(((1,), (1,)), ((), ())),
        preferred_element_type=jnp.float32,
    ) + b_ref[...]
    o0_ref[...] = res[0, :]
    o1_ref[...] = res[1, :]


def _project_table(embed, W, b2d):
    grid = pl.cdiv(_VOCAB, _ROWS_PER_BLK)
    return pl.pallas_call(
        _project_body,
        grid=(grid,),
        in_specs=[
            pl.BlockSpec((_ROWS_PER_BLK, _EMB), lambda i: (i, 0)),
            pl.BlockSpec((_CLS, _EMB), lambda i: (0, 0)),
            pl.BlockSpec((_CLS, 1), lambda i: (0, 0)),
        ],
        out_specs=[
            pl.BlockSpec((_ROWS_PER_BLK,), lambda i: (i,)),
            pl.BlockSpec((_ROWS_PER_BLK,), lambda i: (i,)),
        ],
        out_shape=[
            jax.ShapeDtypeStruct((_VOCAB,), jnp.float32),
            jax.ShapeDtypeStruct((_VOCAB,), jnp.float32),
        ],
    )(embed, W, b2d)


_CHUNK = 640


def _make_gather(n_total):
    info = plsc.get_sparse_core_info()
    nc, ns = info.num_cores, info.num_subcores
    nw = nc * ns
    per_w = n_total // nw
    steps = per_w // _CHUNK
    v_stage = _VOCAB // (ns // 2)
    mesh = plsc.VectorSubcoreMesh(core_axis_name="c", subcore_axis_name="s")

    @functools.partial(
        pl.kernel,
        out_type=[
            jax.ShapeDtypeStruct((n_total,), jnp.float32),
            jax.ShapeDtypeStruct((n_total,), jnp.float32),
        ],
        mesh=mesh,
        scratch_types=[
            pltpu.VMEM_SHARED((_VOCAB,), jnp.float32),
            pltpu.VMEM_SHARED((_VOCAB,), jnp.float32),
            pltpu.VMEM((_CHUNK,), jnp.int32),
            pltpu.VMEM((_CHUNK,), jnp.float32),
            pltpu.VMEM((_CHUNK,), jnp.float32),
            pltpu.SemaphoreType.DMA,
            pltpu.SemaphoreType.DMA,
        ],
        compiler_params=pltpu.CompilerParams(use_tc_tiling_on_sc=False,
                                             needs_layout_passes=False),
    )
    def gather_kernel(p0_hbm, p1_hbm, idx_hbm, o0_hbm, o1_hbm,
                      sh0, sh1, idx_v, v0, v1, sem, sem_o):
        cid = lax.axis_index("c")
        sid = lax.axis_index("s")
        wid = sid * nc + cid
        base = wid * per_w

        @pl.when(sid < ns // 2)
        def _stage0():
            pltpu.sync_copy(p0_hbm.at[pl.ds(sid * v_stage, v_stage)],
                            sh0.at[pl.ds(sid * v_stage, v_stage)])

        @pl.when(sid >= ns // 2)
        def _stage1():
            s2 = sid - ns // 2
            pltpu.sync_copy(p1_hbm.at[pl.ds(s2 * v_stage, v_stage)],
                            sh1.at[pl.ds(s2 * v_stage, v_stage)])

        plsc.subcore_barrier()

        def step(j, carry):
            off = base + j * _CHUNK
            pltpu.sync_copy(idx_hbm.at[pl.ds(off, _CHUNK)], idx_v)

            @pl.when(j > 0)
            def _drain_prev():
                pltpu.make_async_copy(
                    v0, o0_hbm.at[pl.ds(off, _CHUNK)], sem_o).wait()
                pltpu.make_async_copy(
                    v1, o1_hbm.at[pl.ds(off, _CHUNK)], sem_o).wait()

            c0 = pltpu.async_copy(sh0.at[idx_v], v0, sem)
            c1 = pltpu.async_copy(sh1.at[idx_v], v1, sem)
            c0.wait()
            c1.wait()
            pltpu.async_copy(v0, o0_hbm.at[pl.ds(off, _CHUNK)], sem_o)
            pltpu.async_copy(v1, o1_hbm.at[pl.ds(off, _CHUNK)], sem_o)
            return carry

        lax.fori_loop(0, steps, step, 0)
        last = base + (steps - 1) * _CHUNK
        pltpu.make_async_copy(
            v0, o0_hbm.at[pl.ds(last, _CHUNK)], sem_o).wait()
        pltpu.make_async_copy(
            v1, o1_hbm.at[pl.ds(last, _CHUNK)], sem_o).wait()

    return gather_kernel


def kernel(x, embed, W, b):
    B, L = x.shape
    p0, p1 = _project_table(embed, W, b.reshape(_CLS, 1))
    o0, o1 = _make_gather(B * L)(p0, p1, x.reshape(-1))
    return jnp.stack([o0, o1], axis=-1).reshape(B, L, _CLS)


# speedup vs baseline: 1.2395x; 1.0791x over previous
import functools

import jax
import jax.numpy as jnp
from jax import lax
from jax.experimental import pallas as pl
from jax.experimental.pallas import tpu as pltpu
from jax.experimental.pallas import tpu_sc as plsc

_VOCAB = 1000000
_EMB = 16
_CLS = 2

_ROWS_PER_BLK = 32768


def _project_body(e_ref, w_ref, b_ref, o0_ref, o1_ref):
    res = lax.dot_general(
        w_ref[...], e_ref[...],
        dimension_numbers=(((1,), (1,)), ((), ())),
        preferred_element_type=jnp.float32,
    ) + b_ref[...]
    o0_ref[...] = res[0, :]
    o1_ref[...] = res[1, :]


def _project_table(embed, W, b2d):
    grid = pl.cdiv(_VOCAB, _ROWS_PER_BLK)
    return pl.pallas_call(
        _project_body,
        grid=(grid,),
        in_specs=[
            pl.BlockSpec((_ROWS_PER_BLK, _EMB), lambda i: (i, 0)),
            pl.BlockSpec((_CLS, _EMB), lambda i: (0, 0)),
            pl.BlockSpec((_CLS, 1), lambda i: (0, 0)),
        ],
        out_specs=[
            pl.BlockSpec((_ROWS_PER_BLK,), lambda i: (i,)),
            pl.BlockSpec((_ROWS_PER_BLK,), lambda i: (i,)),
        ],
        out_shape=[
            jax.ShapeDtypeStruct((_VOCAB,), jnp.float32),
            jax.ShapeDtypeStruct((_VOCAB,), jnp.float32),
        ],
    )(embed, W, b2d)


_CHUNK = 512


def _make_gather(n_total):
    info = plsc.get_sparse_core_info()
    nc, ns = info.num_cores, info.num_subcores
    nw = nc * ns
    per_w = n_total // nw
    steps = per_w // _CHUNK
    v_stage = _VOCAB // (ns // 2)
    mesh = plsc.VectorSubcoreMesh(core_axis_name="c", subcore_axis_name="s")

    @functools.partial(
        pl.kernel,
        out_type=[
            jax.ShapeDtypeStruct((n_total,), jnp.float32),
            jax.ShapeDtypeStruct((n_total,), jnp.float32),
        ],
        mesh=mesh,
        scratch_types=[
            pltpu.VMEM_SHARED((_VOCAB,), jnp.float32),
            pltpu.VMEM_SHARED((_VOCAB,), jnp.float32),
            pltpu.VMEM((_CHUNK,), jnp.int32),
            pltpu.VMEM((_CHUNK,), jnp.int32),
            pltpu.VMEM((_CHUNK,), jnp.float32),
            pltpu.VMEM((_CHUNK,), jnp.float32),
            pltpu.SemaphoreType.DMA,
            pltpu.SemaphoreType.DMA,
            pltpu.SemaphoreType.DMA,
        ],
        compiler_params=pltpu.CompilerParams(use_tc_tiling_on_sc=False,
                                             needs_layout_passes=False),
    )
    def gather_kernel(p0_hbm, p1_hbm, idx_hbm, o0_hbm, o1_hbm,
                      sh0, sh1, idx_a, idx_b, v0, v1, sem, sem_o, sem_i):
        cid = lax.axis_index("c")
        sid = lax.axis_index("s")
        wid = sid * nc + cid
        base = wid * per_w

        @pl.when(sid < ns // 2)
        def _stage0():
            pltpu.sync_copy(p0_hbm.at[pl.ds(sid * v_stage, v_stage)],
                            sh0.at[pl.ds(sid * v_stage, v_stage)])

        @pl.when(sid >= ns // 2)
        def _stage1():
            s2 = sid - ns // 2
            pltpu.sync_copy(p1_hbm.at[pl.ds(s2 * v_stage, v_stage)],
                            sh1.at[pl.ds(s2 * v_stage, v_stage)])

        plsc.subcore_barrier()

        pairs = steps // 2
        pltpu.async_copy(idx_hbm.at[pl.ds(base, _CHUNK)], idx_a, sem_i)

        def drain_prev_outs(off):
            pltpu.make_async_copy(
                v0, o0_hbm.at[pl.ds(off, _CHUNK)], sem_o).wait()
            pltpu.make_async_copy(
                v1, o1_hbm.at[pl.ds(off, _CHUNK)], sem_o).wait()

        def substep(idx_v, off, first):
            @pl.when(jnp.logical_not(first))
            def _():
                drain_prev_outs(off)
            c0 = pltpu.async_copy(sh0.at[idx_v], v0, sem)
            c1 = pltpu.async_copy(sh1.at[idx_v], v1, sem)
            c0.wait()
            c1.wait()
            pltpu.async_copy(v0, o0_hbm.at[pl.ds(off, _CHUNK)], sem_o)
            pltpu.async_copy(v1, o1_hbm.at[pl.ds(off, _CHUNK)], sem_o)

        def pair(jj, carry):
            off0 = base + (jj * 2) * _CHUNK
            off1 = off0 + _CHUNK
            pltpu.make_async_copy(
                idx_hbm.at[pl.ds(off0, _CHUNK)], idx_a, sem_i).wait()
            pltpu.async_copy(idx_hbm.at[pl.ds(off1, _CHUNK)], idx_b, sem_i)
            substep(idx_a, off0, jj == 0)

            pltpu.make_async_copy(
                idx_hbm.at[pl.ds(off1, _CHUNK)], idx_b, sem_i).wait()

            @pl.when(jj < pairs - 1)
            def _prefetch_next():
                pltpu.async_copy(
                    idx_hbm.at[pl.ds(off1 + _CHUNK, _CHUNK)], idx_a, sem_i)

            substep(idx_b, off1, False)
            return carry

        lax.fori_loop(0, pairs, pair, 0)
        drain_prev_outs(base)

    return gather_kernel


def kernel(x, embed, W, b):
    B, L = x.shape
    p0, p1 = _project_table(embed, W, b.reshape(_CLS, 1))
    o0, o1 = _make_gather(B * L)(p0, p1, x.reshape(-1))
    return jnp.stack([o0, o1], axis=-1).reshape(B, L, _CLS)
